# baseline (speedup 1.0000x reference)
import functools

import jax
import jax.numpy as jnp
from jax import lax
from jax.experimental import pallas as pl
from jax.experimental.pallas import tpu as pltpu
from jax.experimental.pallas import tpu_sc as plsc

EMB = 128
CH = 128
NB = 4

_GATHER_DNUMS = lax.GatherDimensionNumbers(
    offset_dims=(), collapsed_slice_dims=(0,), start_index_map=(0,)
)


@functools.lru_cache(maxsize=None)
def _build_sc_compute(b_flat: int):
    info = plsc.get_sparse_core_info()
    nc, ns, nl = info.num_cores, info.num_subcores, info.num_lanes
    nw = nc * ns
    rows_per_w = b_flat // nw
    n_ch = rows_per_w // CH
    n_outer = n_ch // NB
    assert rows_per_w % CH == 0 and n_ch % NB == 0
    mesh = plsc.VectorSubcoreMesh(core_axis_name="c", subcore_axis_name="s")

    @functools.partial(
        pl.kernel,
        mesh=mesh,
        out_type=jax.ShapeDtypeStruct((b_flat * EMB,), jnp.float32),
        scratch_types=[
            pltpu.VMEM((640,), jnp.float32),
            pltpu.VMEM((rows_per_w,), jnp.int32),
            pltpu.VMEM((NB * CH * EMB,), jnp.float32),
        ]
        + [pltpu.SemaphoreType.DMA] * NB,
    )
    def k(coef_hbm, x_hbm, out_hbm, coef_v, xi_v, rows_v, *sem_o):
        wid = lax.axis_index("s") * nc + lax.axis_index("c")
        base = wid * rows_per_w
        pltpu.sync_copy(coef_hbm, coef_v)
        pltpu.sync_copy(x_hbm.at[pl.ds(base, rows_per_w)], xi_v)
        st1 = [coef_v[pl.ds(j * nl, nl)] for j in range(8)]
        dP = [coef_v[pl.ds(128 + j * nl, nl)] for j in range(8)]
        bH = [coef_v[pl.ds(256 + j * nl, nl)] for j in range(8)]
        bV = [coef_v[pl.ds(384 + j * nl, nl)] for j in range(8)]
        ac1 = [coef_v[pl.ds(512 + j * nl, nl)] for j in range(8)]

        def out_copy(c, b, start):
            src = rows_v.at[pl.ds(b * CH * EMB, CH * EMB)]
            dst = out_hbm.at[pl.ds((base + c * CH) * EMB, CH * EMB)]
            if start:
                return pltpu.async_copy(src, dst, sem_o[b])
            return pltpu.make_async_copy(src, dst, sem_o[b]).wait()

        def group(c, b, g):
            x16 = xi_v[pl.ds(c * CH + g * nl, nl)].astype(jnp.float32)
            for kk in range(nl):
                m = lax.gather(
                    x16,
                    jnp.full((nl, 1), kk, jnp.int32),
                    _GATHER_DNUMS,
                    slice_sizes=(1,),
                    mode=lax.GatherScatterMode.PROMISE_IN_BOUNDS,
                )
                seg = kk % 4
                off = (b * CH + g * nl + kk) * EMB
                for j in range(8):
                    if seg == 0:
                        row = m * st1[j]
                    elif seg == 1:
                        row = bH[j] + m * dP[j]
                    elif seg == 2:
                        row = bV[j] + m * dP[j]
                    else:
                        row = m * ac1[j]
                    rows_v[pl.ds(off + j * nl, nl)] = row

        for c in range(n_ch):
            b = c % NB
            if c >= NB:
                out_copy(c - NB, b, start=False)

            def gbody(g, carry2, c=c, b=b):
                group(c, b, g)
                return carry2

            lax.fori_loop(0, CH // nl, gbody, 0)
            out_copy(c, b, start=True)
        for c in range(n_ch - NB, n_ch):
            out_copy(c, c % NB, start=False)

    return k


def kernel(x, street_table, action_table, position_table, order_table):
    batch = x.shape[0]
    coef = jnp.concatenate(
        (
            street_table[1],
            position_table[1] - position_table[0],
            position_table[0] + order_table[0],
            position_table[0] + order_table[1],
            action_table[1],
        )
    )
    x_flat = x.astype(jnp.int32).reshape(-1)
    out = _build_sc_compute(4 * batch)(coef, x_flat)
    return out.reshape(batch, 4 * EMB)


# speedup vs baseline: 1.0225x; 1.0225x over previous
import functools

import jax
import jax.numpy as jnp
from jax import lax
from jax.experimental import pallas as pl
from jax.experimental.pallas import tpu as pltpu
from jax.experimental.pallas import tpu_sc as plsc

EMB = 128
CH = 256
NB = 3

_GATHER_DNUMS = lax.GatherDimensionNumbers(
    offset_dims=(), collapsed_slice_dims=(0,), start_index_map=(0,)
)


@functools.lru_cache(maxsize=None)
def _build_sc_compute(b_flat: int):
    info = plsc.get_sparse_core_info()
    nc, ns, nl = info.num_cores, info.num_subcores, info.num_lanes
    nw = nc * ns
    rows_per_w = b_flat // nw
    n_ch = rows_per_w // CH
    n_outer = n_ch // NB
    assert rows_per_w % CH == 0 and n_ch >= NB
    mesh = plsc.VectorSubcoreMesh(core_axis_name="c", subcore_axis_name="s")

    @functools.partial(
        pl.kernel,
        mesh=mesh,
        out_type=jax.ShapeDtypeStruct((b_flat * EMB,), jnp.float32),
        scratch_types=[
            pltpu.VMEM((640,), jnp.float32),
            pltpu.VMEM((rows_per_w,), jnp.int32),
            pltpu.VMEM((NB * CH * EMB,), jnp.float32),
        ]
        + [pltpu.SemaphoreType.DMA] * NB,
    )
    def k(coef_hbm, x_hbm, out_hbm, coef_v, xi_v, rows_v, *sem_o):
        wid = lax.axis_index("s") * nc + lax.axis_index("c")
        base = wid * rows_per_w
        pltpu.sync_copy(coef_hbm, coef_v)
        pltpu.sync_copy(x_hbm.at[pl.ds(base, rows_per_w)], xi_v)
        st1 = [coef_v[pl.ds(j * nl, nl)] for j in range(8)]
        dP = [coef_v[pl.ds(128 + j * nl, nl)] for j in range(8)]
        bH = [coef_v[pl.ds(256 + j * nl, nl)] for j in range(8)]
        bV = [coef_v[pl.ds(384 + j * nl, nl)] for j in range(8)]
        ac1 = [coef_v[pl.ds(512 + j * nl, nl)] for j in range(8)]

        def out_copy(c, b, start):
            src = rows_v.at[pl.ds(b * CH * EMB, CH * EMB)]
            dst = out_hbm.at[pl.ds((base + c * CH) * EMB, CH * EMB)]
            if start:
                return pltpu.async_copy(src, dst, sem_o[b])
            return pltpu.make_async_copy(src, dst, sem_o[b]).wait()

        def group(c, b, g):
            x16 = xi_v[pl.ds(c * CH + g * nl, nl)].astype(jnp.float32)
            for kk in range(nl):
                m = lax.gather(
                    x16,
                    jnp.full((nl, 1), kk, jnp.int32),
                    _GATHER_DNUMS,
                    slice_sizes=(1,),
                    mode=lax.GatherScatterMode.PROMISE_IN_BOUNDS,
                )
                seg = kk % 4
                off = (b * CH + g * nl + kk) * EMB
                for j in range(8):
                    if seg == 0:
                        row = m * st1[j]
                    elif seg == 1:
                        row = bH[j] + m * dP[j]
                    elif seg == 2:
                        row = bV[j] + m * dP[j]
                    else:
                        row = m * ac1[j]
                    rows_v[pl.ds(off + j * nl, nl)] = row

        for c in range(n_ch):
            b = c % NB
            if c >= NB:
                out_copy(c - NB, b, start=False)

            def gbody(g, carry2, c=c, b=b):
                group(c, b, g)
                return carry2

            lax.fori_loop(0, CH // nl, gbody, 0)
            out_copy(c, b, start=True)
        for c in range(n_ch - NB, n_ch):
            out_copy(c, c % NB, start=False)

    return k


def kernel(x, street_table, action_table, position_table, order_table):
    batch = x.shape[0]
    coef = jnp.concatenate(
        (
            street_table[1],
            position_table[1] - position_table[0],
            position_table[0] + order_table[0],
            position_table[0] + order_table[1],
            action_table[1],
        )
    )
    x_flat = x.astype(jnp.int32).reshape(-1)
    out = _build_sc_compute(4 * batch)(coef, x_flat)
    return out.reshape(batch, 4 * EMB)


# speedup vs baseline: 1.0228x; 1.0002x over previous
import functools

import jax
import jax.numpy as jnp
from jax import lax
from jax.experimental import pallas as pl
from jax.experimental.pallas import tpu as pltpu
from jax.experimental.pallas import tpu_sc as plsc

EMB = 128
CH = 256
NB = 2

_GATHER_DNUMS = lax.GatherDimensionNumbers(
    offset_dims=(), collapsed_slice_dims=(0,), start_index_map=(0,)
)


@functools.lru_cache(maxsize=None)
def _build_sc_compute(b_flat: int):
    info = plsc.get_sparse_core_info()
    nc, ns, nl = info.num_cores, info.num_subcores, info.num_lanes
    nw = nc * ns
    rows_per_w = b_flat // nw
    n_ch = rows_per_w // CH
    n_outer = n_ch // NB
    assert rows_per_w % CH == 0 and n_ch >= NB
    mesh = plsc.VectorSubcoreMesh(core_axis_name="c", subcore_axis_name="s")

    @functools.partial(
        pl.kernel,
        mesh=mesh,
        out_type=jax.ShapeDtypeStruct((b_flat * EMB,), jnp.float32),
        scratch_types=[
            pltpu.VMEM((640,), jnp.float32),
            pltpu.VMEM((rows_per_w,), jnp.int32),
            pltpu.VMEM((NB * CH * EMB,), jnp.float32),
        ]
        + [pltpu.SemaphoreType.DMA] * NB,
    )
    def k(coef_hbm, x_hbm, out_hbm, coef_v, xi_v, rows_v, *sem_o):
        wid = lax.axis_index("s") * nc + lax.axis_index("c")
        base = wid * rows_per_w
        pltpu.sync_copy(coef_hbm, coef_v)
        pltpu.sync_copy(x_hbm.at[pl.ds(base, rows_per_w)], xi_v)
        st1 = [coef_v[pl.ds(j * nl, nl)] for j in range(8)]
        dP = [coef_v[pl.ds(128 + j * nl, nl)] for j in range(8)]
        bH = [coef_v[pl.ds(256 + j * nl, nl)] for j in range(8)]
        bV = [coef_v[pl.ds(384 + j * nl, nl)] for j in range(8)]
        ac1 = [coef_v[pl.ds(512 + j * nl, nl)] for j in range(8)]

        def out_copy(c, b, start):
            src = rows_v.at[pl.ds(b * CH * EMB, CH * EMB)]
            dst = out_hbm.at[pl.ds((base + c * CH) * EMB, CH * EMB)]
            if start:
                return pltpu.async_copy(src, dst, sem_o[b])
            return pltpu.make_async_copy(src, dst, sem_o[b]).wait()

        def group(c, b, g):
            x16 = xi_v[pl.ds(c * CH + g * nl, nl)].astype(jnp.float32)
            for kk in range(nl):
                m = lax.gather(
                    x16,
                    jnp.full((nl, 1), kk, jnp.int32),
                    _GATHER_DNUMS,
                    slice_sizes=(1,),
                    mode=lax.GatherScatterMode.PROMISE_IN_BOUNDS,
                )
                seg = kk % 4
                off = (b * CH + g * nl + kk) * EMB
                for j in range(8):
                    if seg == 0:
                        row = m * st1[j]
                    elif seg == 1:
                        row = bH[j] + m * dP[j]
                    elif seg == 2:
                        row = bV[j] + m * dP[j]
                    else:
                        row = m * ac1[j]
                    rows_v[pl.ds(off + j * nl, nl)] = row

        for c in range(n_ch):
            b = c % NB
            if c >= NB:
                out_copy(c - NB, b, start=False)

            def gbody(g, carry2, c=c, b=b):
                group(c, b, g)
                return carry2

            lax.fori_loop(0, CH // nl, gbody, 0)
            out_copy(c, b, start=True)
        for c in range(n_ch - NB, n_ch):
            out_copy(c, c % NB, start=False)

    return k


def kernel(x, street_table, action_table, position_table, order_table):
    batch = x.shape[0]
    coef = jnp.concatenate(
        (
            street_table[1],
            position_table[1] - position_table[0],
            position_table[0] + order_table[0],
            position_table[0] + order_table[1],
            action_table[1],
        )
    )
    x_flat = x.astype(jnp.int32).reshape(-1)
    out = _build_sc_compute(4 * batch)(coef, x_flat)
    return out.reshape(batch, 4 * EMB)


# speedup vs baseline: 1.8119x; 1.7716x over previous
import functools

import jax
import jax.numpy as jnp
from jax import lax
from jax.experimental import pallas as pl
from jax.experimental.pallas import tpu as pltpu
from jax.experimental.pallas import tpu_sc as plsc

EMB = 128
CH = 256
NB = 2

_GATHER_DNUMS = lax.GatherDimensionNumbers(
    offset_dims=(), collapsed_slice_dims=(0,), start_index_map=(0,)
)


@functools.lru_cache(maxsize=None)
def _build_sc_compute(b_flat: int):
    info = plsc.get_sparse_core_info()
    nc, ns, nl = info.num_cores, info.num_subcores, info.num_lanes
    nw = nc * ns
    rows_per_w = b_flat // nw
    n_ch = rows_per_w // CH
    n_outer = n_ch // NB
    assert rows_per_w % CH == 0 and n_ch >= NB
    mesh = plsc.VectorSubcoreMesh(core_axis_name="c", subcore_axis_name="s")

    @functools.partial(
        pl.kernel,
        mesh=mesh,
        out_type=jax.ShapeDtypeStruct((b_flat // 4, 4 * EMB), jnp.float32),
        scratch_types=[
            pltpu.VMEM((640,), jnp.float32),
            pltpu.VMEM((rows_per_w,), jnp.int32),
            pltpu.VMEM((NB, CH // 4, 4 * EMB), jnp.float32),
        ]
        + [pltpu.SemaphoreType.DMA] * NB,
    )
    def k(coef_hbm, x_hbm, out_hbm, coef_v, xi_v, rows_v, *sem_o):
        wid = lax.axis_index("s") * nc + lax.axis_index("c")
        base = wid * rows_per_w
        pltpu.sync_copy(coef_hbm, coef_v)
        pltpu.sync_copy(x_hbm.at[pl.ds(base, rows_per_w)], xi_v)
        st1 = [coef_v[pl.ds(j * nl, nl)] for j in range(8)]
        dP = [coef_v[pl.ds(128 + j * nl, nl)] for j in range(8)]
        bH = [coef_v[pl.ds(256 + j * nl, nl)] for j in range(8)]
        bV = [coef_v[pl.ds(384 + j * nl, nl)] for j in range(8)]
        ac1 = [coef_v[pl.ds(512 + j * nl, nl)] for j in range(8)]

        def out_copy(c, b, start):
            src = rows_v.at[b]
            dst = out_hbm.at[pl.ds(pl.multiple_of((base + c * CH) // 4, CH // 4), CH // 4)]
            if start:
                return pltpu.async_copy(src, dst, sem_o[b])
            return pltpu.make_async_copy(src, dst, sem_o[b]).wait()

        def group(c, b, g):
            x16 = xi_v[pl.ds(c * CH + g * nl, nl)].astype(jnp.float32)
            for kk in range(nl):
                m = lax.gather(
                    x16,
                    jnp.full((nl, 1), kk, jnp.int32),
                    _GATHER_DNUMS,
                    slice_sizes=(1,),
                    mode=lax.GatherScatterMode.PROMISE_IN_BOUNDS,
                )
                seg = kk % 4
                sub = g * 4 + kk // 4
                for j in range(8):
                    if seg == 0:
                        row = m * st1[j]
                    elif seg == 1:
                        row = bH[j] + m * dP[j]
                    elif seg == 2:
                        row = bV[j] + m * dP[j]
                    else:
                        row = m * ac1[j]
                    rows_v[b, sub, pl.ds(seg * EMB + j * nl, nl)] = row

        for c in range(n_ch):
            b = c % NB
            if c >= NB:
                out_copy(c - NB, b, start=False)

            def gbody(g, carry2, c=c, b=b):
                group(c, b, g)
                return carry2

            lax.fori_loop(0, CH // nl, gbody, 0)
            out_copy(c, b, start=True)
        for c in range(n_ch - NB, n_ch):
            out_copy(c, c % NB, start=False)

    return k


def kernel(x, street_table, action_table, position_table, order_table):
    batch = x.shape[0]
    coef = jnp.concatenate(
        (
            street_table[1],
            position_table[1] - position_table[0],
            position_table[0] + order_table[0],
            position_table[0] + order_table[1],
            action_table[1],
        )
    )
    x_flat = x.astype(jnp.int32).reshape(-1)
    return _build_sc_compute(4 * batch)(coef, x_flat)


# speedup vs baseline: 2.4927x; 1.3758x over previous
import functools

import jax
import jax.numpy as jnp
from jax import lax
from jax.experimental import pallas as pl
from jax.experimental.pallas import tpu as pltpu
from jax.experimental.pallas import tpu_sc as plsc

EMB = 128
BCH = 64
NB = 2
SG = 16

_GATHER_DNUMS = lax.GatherDimensionNumbers(
    offset_dims=(), collapsed_slice_dims=(0,), start_index_map=(0,)
)


@functools.lru_cache(maxsize=None)
def _build_sc_compute(batch: int):
    info = plsc.get_sparse_core_info()
    nc, ns, nl = info.num_cores, info.num_subcores, info.num_lanes
    nw = nc * ns
    rows_per_w = batch // nw
    n_ch = rows_per_w // BCH
    assert rows_per_w % BCH == 0 and n_ch >= NB and BCH % SG == 0
    mesh = plsc.VectorSubcoreMesh(core_axis_name="c", subcore_axis_name="s")

    @functools.partial(
        pl.kernel,
        mesh=mesh,
        out_type=jax.ShapeDtypeStruct((batch, 4 * EMB), jnp.float32),
        scratch_types=[
            pltpu.VMEM((640,), jnp.float32),
            pltpu.VMEM((rows_per_w,), jnp.int32),
            pltpu.VMEM((NB, BCH, 4 * EMB), jnp.float32),
        ]
        + [pltpu.SemaphoreType.DMA] * NB,
    )
    def k(coef_hbm, mc_hbm, out_hbm, coef_v, mc_v, rows_v, *sem_o):
        wid = lax.axis_index("s") * nc + lax.axis_index("c")
        base = wid * rows_per_w
        pltpu.sync_copy(coef_hbm, coef_v)
        pltpu.sync_copy(mc_hbm.at[pl.ds(base, rows_per_w)], mc_v)
        st1 = [coef_v[pl.ds(j * nl, nl)] for j in range(8)]
        dP = [coef_v[pl.ds(128 + j * nl, nl)] for j in range(8)]
        bH = [coef_v[pl.ds(256 + j * nl, nl)] for j in range(8)]
        bV = [coef_v[pl.ds(384 + j * nl, nl)] for j in range(8)]
        ac1 = [coef_v[pl.ds(512 + j * nl, nl)] for j in range(8)]

        def out_copy(c, b, start):
            src = rows_v.at[b]
            dst = out_hbm.at[pl.ds(pl.multiple_of(base + c * BCH, BCH), BCH)]
            if start:
                return pltpu.async_copy(src, dst, sem_o[b])
            return pltpu.make_async_copy(src, dst, sem_o[b]).wait()

        def supergroup(c, b, sg):
            mc16 = mc_v[pl.ds(c * BCH + sg * SG, SG)]
            for kk in range(SG):
                mc = lax.gather(
                    mc16,
                    jnp.full((nl, 1), kk, jnp.int32),
                    _GATHER_DNUMS,
                    slice_sizes=(1,),
                    mode=lax.GatherScatterMode.PROMISE_IN_BOUNDS,
                )
                sub = sg * SG + kk
                for seg in range(4):
                    bit = mc >> seg if seg else mc
                    mf = (bit & 1).astype(jnp.float32)
                    for j in range(8):
                        if seg == 0:
                            row = mf * st1[j]
                        elif seg == 1:
                            row = bH[j] + mf * dP[j]
                        elif seg == 2:
                            row = bV[j] + mf * dP[j]
                        else:
                            row = mf * ac1[j]
                        rows_v[b, sub, pl.ds(seg * EMB + j * nl, nl)] = row

        def chunk_body(s, carry):
            for b in range(NB):
                c = s * NB + b

                @pl.when(s > 0)
                def _():
                    out_copy(c - NB, b, start=False)

                def gbody(sg, carry2, c=c, b=b):
                    supergroup(c, b, sg)
                    return carry2

                lax.fori_loop(0, BCH // SG, gbody, 0)
                out_copy(c, b, start=True)
            return carry

        lax.fori_loop(0, n_ch // NB, chunk_body, 0)
        for c in range(n_ch - NB, n_ch):
            out_copy(c, c % NB, start=False)

    return k


def kernel(x, street_table, action_table, position_table, order_table):
    batch = x.shape[0]
    coef = jnp.concatenate(
        (
            street_table[1],
            position_table[1] - position_table[0],
            position_table[0] + order_table[0],
            position_table[0] + order_table[1],
            action_table[1],
        )
    )
    weights = jnp.array([1, 2, 4, 8], dtype=jnp.int32)
    mcode = jnp.sum(x.astype(jnp.int32) * weights, axis=1, dtype=jnp.int32)
    return _build_sc_compute(batch)(coef, mcode)


# speedup vs baseline: 2.5547x; 1.0248x over previous
import functools

import jax
import jax.numpy as jnp
from jax import lax
from jax.experimental import pallas as pl
from jax.experimental.pallas import tpu as pltpu
from jax.experimental.pallas import tpu_sc as plsc

EMB = 128
BCH = 64
NB = 2
SG = 16

_GATHER_DNUMS = lax.GatherDimensionNumbers(
    offset_dims=(), collapsed_slice_dims=(0,), start_index_map=(0,)
)


@functools.lru_cache(maxsize=None)
def _build_sc_compute(batch: int):
    info = plsc.get_sparse_core_info()
    nc, ns, nl = info.num_cores, info.num_subcores, info.num_lanes
    nw = nc * ns
    rows_per_w = batch // nw
    n_ch = rows_per_w // BCH
    assert rows_per_w % BCH == 0 and n_ch >= NB and BCH % SG == 0
    mesh = plsc.VectorSubcoreMesh(core_axis_name="c", subcore_axis_name="s")

    @functools.partial(
        pl.kernel,
        mesh=mesh,
        out_type=jax.ShapeDtypeStruct((batch, 4 * EMB), jnp.float32),
        scratch_types=[
            pltpu.VMEM((640,), jnp.float32),
            pltpu.VMEM((rows_per_w,), jnp.int32),
            pltpu.VMEM((NB, BCH, 4 * EMB), jnp.float32),
        ]
        + [pltpu.SemaphoreType.DMA] * NB,
    )
    def k(coef_hbm, mc_hbm, out_hbm, coef_v, mc_v, rows_v, *sem_o):
        wid = lax.axis_index("s") * nc + lax.axis_index("c")
        base = wid * rows_per_w
        pltpu.sync_copy(coef_hbm, coef_v)
        pltpu.sync_copy(mc_hbm.at[pl.ds(base, rows_per_w)], mc_v)
        st1 = [coef_v[pl.ds(j * nl, nl)] for j in range(8)]
        dP = [coef_v[pl.ds(128 + j * nl, nl)] for j in range(8)]
        bH = [coef_v[pl.ds(256 + j * nl, nl)] for j in range(8)]
        bV = [coef_v[pl.ds(384 + j * nl, nl)] for j in range(8)]
        ac1 = [coef_v[pl.ds(512 + j * nl, nl)] for j in range(8)]

        def out_copy(c, b, start):
            src = rows_v.at[b]
            dst = out_hbm.at[pl.ds(pl.multiple_of(base + c * BCH, BCH), BCH)]
            if start:
                return pltpu.async_copy(src, dst, sem_o[b])
            return pltpu.make_async_copy(src, dst, sem_o[b]).wait()

        def supergroup(c, b, sg):
            mc16 = mc_v[pl.ds(c * BCH + sg * SG, SG)]

            def row_body(kk, carry3):
                mc = lax.gather(
                    mc16,
                    jnp.broadcast_to(kk, (nl, 1)).astype(jnp.int32),
                    _GATHER_DNUMS,
                    slice_sizes=(1,),
                    mode=lax.GatherScatterMode.PROMISE_IN_BOUNDS,
                )
                sub = sg * SG + kk
                for seg in range(4):
                    bit = mc >> seg if seg else mc
                    mf = (bit & 1).astype(jnp.float32)
                    for j in range(8):
                        if seg == 0:
                            row = mf * st1[j]
                        elif seg == 1:
                            row = bH[j] + mf * dP[j]
                        elif seg == 2:
                            row = bV[j] + mf * dP[j]
                        else:
                            row = mf * ac1[j]
                        rows_v[b, sub, pl.ds(seg * EMB + j * nl, nl)] = row
                return carry3

            lax.fori_loop(0, SG, row_body, 0)

        def chunk_body(s, carry):
            for b in range(NB):
                c = s * NB + b

                @pl.when(s > 0)
                def _():
                    out_copy(c - NB, b, start=False)

                def gbody(sg, carry2, c=c, b=b):
                    supergroup(c, b, sg)
                    return carry2

                lax.fori_loop(0, BCH // SG, gbody, 0)
                out_copy(c, b, start=True)
            return carry

        lax.fori_loop(0, n_ch // NB, chunk_body, 0)
        for c in range(n_ch - NB, n_ch):
            out_copy(c, c % NB, start=False)

    return k


def kernel(x, street_table, action_table, position_table, order_table):
    batch = x.shape[0]
    coef = jnp.concatenate(
        (
            street_table[1],
            position_table[1] - position_table[0],
            position_table[0] + order_table[0],
            position_table[0] + order_table[1],
            action_table[1],
        )
    )
    weights = jnp.array([1, 2, 4, 8], dtype=jnp.int32)
    mcode = jnp.sum(x.astype(jnp.int32) * weights, axis=1, dtype=jnp.int32)
    return _build_sc_compute(batch)(coef, mcode)


# speedup vs baseline: 2.5972x; 1.0166x over previous
import functools

import jax
import jax.numpy as jnp
from jax import lax
from jax.experimental import pallas as pl
from jax.experimental.pallas import tpu as pltpu
from jax.experimental.pallas import tpu_sc as plsc

EMB = 128
BCH = 64
NB = 2
SG = 16

_GATHER_DNUMS = lax.GatherDimensionNumbers(
    offset_dims=(), collapsed_slice_dims=(0,), start_index_map=(0,)
)


@functools.lru_cache(maxsize=None)
def _build_sc_compute(batch: int):
    info = plsc.get_sparse_core_info()
    nc, ns, nl = info.num_cores, info.num_subcores, info.num_lanes
    nw = nc * ns
    rows_per_w = batch // nw
    n_ch = rows_per_w // BCH
    assert rows_per_w % BCH == 0 and n_ch >= NB and BCH % SG == 0
    mesh = plsc.VectorSubcoreMesh(core_axis_name="c", subcore_axis_name="s")

    @functools.partial(
        pl.kernel,
        mesh=mesh,
        out_type=jax.ShapeDtypeStruct((batch, 4 * EMB), jnp.float32),
        scratch_types=[
            pltpu.VMEM((4, EMB), jnp.float32),
            pltpu.VMEM((6, EMB), jnp.float32),
            pltpu.VMEM((2, EMB), jnp.float32),
            pltpu.VMEM((2, EMB), jnp.float32),
            pltpu.VMEM((rows_per_w,), jnp.int32),
            pltpu.VMEM((NB, BCH, 4 * EMB), jnp.float32),
        ]
        + [pltpu.SemaphoreType.DMA] * NB,
    )
    def k(st_hbm, ac_hbm, po_hbm, od_hbm, mc_hbm, out_hbm,
          st_v, ac_v, po_v, od_v, mc_v, rows_v, *sem_o):
        wid = lax.axis_index("s") * nc + lax.axis_index("c")
        base = wid * rows_per_w
        pltpu.sync_copy(st_hbm, st_v)
        pltpu.sync_copy(ac_hbm, ac_v)
        pltpu.sync_copy(po_hbm, po_v)
        pltpu.sync_copy(od_hbm, od_v)
        pltpu.sync_copy(mc_hbm.at[pl.ds(base, rows_per_w)], mc_v)
        st1 = [st_v[1, pl.ds(j * nl, nl)] for j in range(8)]
        dP = [po_v[1, pl.ds(j * nl, nl)] - po_v[0, pl.ds(j * nl, nl)]
              for j in range(8)]
        bH = [po_v[0, pl.ds(j * nl, nl)] + od_v[0, pl.ds(j * nl, nl)]
              for j in range(8)]
        bV = [po_v[0, pl.ds(j * nl, nl)] + od_v[1, pl.ds(j * nl, nl)]
              for j in range(8)]
        ac1 = [ac_v[1, pl.ds(j * nl, nl)] for j in range(8)]

        def out_copy(c, b, start):
            src = rows_v.at[b]
            dst = out_hbm.at[pl.ds(pl.multiple_of(base + c * BCH, BCH), BCH)]
            if start:
                return pltpu.async_copy(src, dst, sem_o[b])
            return pltpu.make_async_copy(src, dst, sem_o[b]).wait()

        def supergroup(c, b, sg):
            mc16 = mc_v[pl.ds(c * BCH + sg * SG, SG)]

            def row_body(kk, carry3):
                mc = lax.gather(
                    mc16,
                    jnp.broadcast_to(kk, (nl, 1)).astype(jnp.int32),
                    _GATHER_DNUMS,
                    slice_sizes=(1,),
                    mode=lax.GatherScatterMode.PROMISE_IN_BOUNDS,
                )
                sub = sg * SG + kk
                for seg in range(4):
                    bit = mc >> seg if seg else mc
                    mf = (bit & 1).astype(jnp.float32)
                    for j in range(8):
                        if seg == 0:
                            row = mf * st1[j]
                        elif seg == 1:
                            row = bH[j] + mf * dP[j]
                        elif seg == 2:
                            row = bV[j] + mf * dP[j]
                        else:
                            row = mf * ac1[j]
                        rows_v[b, sub, pl.ds(seg * EMB + j * nl, nl)] = row
                return carry3

            lax.fori_loop(0, SG, row_body, 0)

        def chunk_body(s, carry):
            for b in range(NB):
                c = s * NB + b

                @pl.when(s > 0)
                def _():
                    out_copy(c - NB, b, start=False)

                def gbody(sg, carry2, c=c, b=b):
                    supergroup(c, b, sg)
                    return carry2

                lax.fori_loop(0, BCH // SG, gbody, 0)
                out_copy(c, b, start=True)
            return carry

        lax.fori_loop(0, n_ch // NB, chunk_body, 0)
        for c in range(n_ch - NB, n_ch):
            out_copy(c, c % NB, start=False)

    return k


def kernel(x, street_table, action_table, position_table, order_table):
    batch = x.shape[0]
    weights = jnp.array([1, 2, 4, 8], dtype=jnp.int32)
    mcode = jnp.sum(x.astype(jnp.int32) * weights, axis=1, dtype=jnp.int32)
    return _build_sc_compute(batch)(
        street_table, action_table, position_table, order_table, mcode
    )
